# baseline (speedup 1.0000x reference)
import jax
import jax.numpy as jnp
from jax import lax
from jax.experimental import pallas as pl
from jax.experimental.pallas import tpu as pltpu
from jax.experimental.pallas import tpu_sc as plsc

_D = 64
_DP = 128
_SP = 56
_SCALE = float(_D) ** 0.5
_L = 16

_NW = 32
_SEQ = 16384
_SLEN = 50
_SEQ_PER_W = _SEQ // _NW
_CSEQ = 4
_CIDX = _CSEQ * _SLEN
_CHUNKS_PER_W = _SEQ_PER_W // _CSEQ
_IDX_PER_W = _SEQ_PER_W * _SLEN
_NBUF = 2
_G_SPLITS = ((0, 96), (96, 104))


def _gather_body(x_hbm, table_hbm, out_hbm, idx_v, grow_v, stage_v, gsems, ssems):
    wid = lax.axis_index("s") * 2 + lax.axis_index("c")

    pltpu.sync_copy(x_hbm.at[pl.ds(wid * _IDX_PER_W, _IDX_PER_W)], idx_v)

    seq_base = wid * _SEQ_PER_W

    def start_gather(g, b):
        for off, n in _G_SPLITS:
            pltpu.make_async_copy(
                table_hbm.at[idx_v.at[pl.ds(g * _CIDX + off, n)]],
                grow_v.at[b, pl.ds(off, n)],
                gsems.at[b],
            ).start()

    def wait_gather(b):
        for off, n in _G_SPLITS:
            pltpu.make_async_copy(
                table_hbm.at[idx_v.at[pl.ds(off, n)]],
                grow_v.at[b, pl.ds(off, n)],
                gsems.at[b],
            ).wait()

    def start_stores(g, b):
        for k in range(_CSEQ):
            pltpu.make_async_copy(
                stage_v.at[b, k],
                out_hbm.at[seq_base + g * _CSEQ + k, pl.ds(0, _SLEN)],
                ssems.at[b],
            ).start()

    def wait_stores(b):
        for k in range(_CSEQ):
            pltpu.make_async_copy(
                stage_v.at[b, k],
                out_hbm.at[seq_base, pl.ds(0, _SLEN)],
                ssems.at[b],
            ).wait()

    def expand(b):
        for k in range(_CSEQ):
            def body(j, _):
                for c in range(_D // _L):
                    sl = pl.ds(c * _L, _L)
                    stage_v[b, k, j, sl] = grow_v[b, 50 * k + j, sl] * _SCALE
                return 0

            lax.fori_loop(0, _SLEN, body, 0)

    def step(g, b, has_next, need_store_wait):
        wait_gather(b)
        if has_next:
            start_gather(g + 1, 1 - b)
        if need_store_wait:
            wait_stores(b)
        expand(b)
        start_stores(g, b)

    start_gather(0, 0)
    step(0, 0, True, False)
    step(1, 1, True, False)

    def steady(t, _):
        g0 = 2 + t * _NBUF
        step(g0, 0, True, True)
        step(g0 + 1, 1, True, True)
        return 0

    n_groups = (_CHUNKS_PER_W - 2 - 2) // _NBUF
    lax.fori_loop(0, n_groups, steady, 0)

    step(_CHUNKS_PER_W - 2, 0, True, True)
    step(_CHUNKS_PER_W - 1, 1, False, True)

    for b in range(_NBUF):
        wait_stores(b)


@jax.jit
def _embed(x_lin, embed_weight):
    mesh = plsc.VectorSubcoreMesh(core_axis_name="c", subcore_axis_name="s")
    run = pl.kernel(
        _gather_body,
        out_type=jax.ShapeDtypeStruct((_SEQ, _SP, _DP), jnp.float32),
        mesh=mesh,
        scratch_types=[
            pltpu.VMEM((_IDX_PER_W,), jnp.int32),
            pltpu.VMEM((_NBUF, _CIDX, _D), jnp.float32),
            pltpu.VMEM((_NBUF, _CSEQ, _SLEN, _DP), jnp.float32),
            pltpu.SemaphoreType.DMA((_NBUF,)),
            pltpu.SemaphoreType.DMA((_NBUF,)),
        ],
        compiler_params=pltpu.CompilerParams(use_tc_tiling_on_sc=False),
    )
    return run(x_lin, embed_weight)


def kernel(x, embed_weight):
    x_lin = x.reshape(_SEQ * _SLEN).astype(jnp.int32)
    z = _embed(x_lin, embed_weight)
    return z[:, :_SLEN, :_D]


# speedup vs baseline: 1.0170x; 1.0170x over previous
import jax
import jax.numpy as jnp
from jax import lax
from jax.experimental import pallas as pl
from jax.experimental.pallas import tpu as pltpu
from jax.experimental.pallas import tpu_sc as plsc

_D = 64
_DP = 128
_SP = 56
_SCALE = float(_D) ** 0.5

_NW = 32
_SEQ = 16384
_SLEN = 50
_SEQ_PER_W = _SEQ // _NW
_CSEQ = 4
_CIDX = _CSEQ * _SLEN
_CHUNKS_PER_W = _SEQ_PER_W // _CSEQ
_IDX_PER_W = _SEQ_PER_W * _SLEN
_NBUF = 4
_LA = 2
_G_SPLITS = ((0, 96), (96, 104))


def _gather_body(x_hbm, table_hbm, out_hbm, idx_v, grow_v, gsems, ssems):
    wid = lax.axis_index("s") * 2 + lax.axis_index("c")

    pltpu.sync_copy(x_hbm.at[pl.ds(wid * _IDX_PER_W, _IDX_PER_W)], idx_v)

    seq_base = wid * _SEQ_PER_W

    def start_gather(g, b):
        for off, n in _G_SPLITS:
            pltpu.make_async_copy(
                table_hbm.at[idx_v.at[pl.ds(g * _CIDX + off, n)]],
                grow_v.at[b, pl.ds(off, n)],
                gsems.at[b],
            ).start()

    def wait_gather(b):
        for off, n in _G_SPLITS:
            pltpu.make_async_copy(
                table_hbm.at[idx_v.at[pl.ds(off, n)]],
                grow_v.at[b, pl.ds(off, n)],
                gsems.at[b],
            ).wait()

    def start_stores(g, b):
        for k in range(_CSEQ):
            pltpu.make_async_copy(
                grow_v.at[b, pl.ds(k * _SLEN, _SLEN)],
                out_hbm.at[seq_base + g * _CSEQ + k, pl.ds(0, _SLEN),
                           pl.ds(0, _D)],
                ssems.at[b],
            ).start()

    def wait_stores(b):
        for k in range(_CSEQ):
            pltpu.make_async_copy(
                grow_v.at[b, pl.ds(k * _SLEN, _SLEN)],
                out_hbm.at[seq_base, pl.ds(0, _SLEN), pl.ds(0, _D)],
                ssems.at[b],
            ).wait()


    def emit(j, b, g, need_store_wait):
        wait_gather(b)
        start_stores(j, b)
        if g is not None:
            b2 = (b + _LA) % _NBUF
            if need_store_wait:
                wait_stores(b2)
            start_gather(g, b2)

    for g in range(_LA):
        start_gather(g, g % _NBUF)

    for j in range(_NBUF):
        emit(j, j % _NBUF, j + _LA, j + _LA >= _NBUF)

    n_groups = (_CHUNKS_PER_W - _NBUF - _LA) // _NBUF

    def steady(t, _):
        j0 = _NBUF + t * _NBUF
        for i in range(_NBUF):
            emit(j0 + i, i, j0 + i + _LA, True)
        return 0

    lax.fori_loop(0, n_groups, steady, 0)

    for j in range(_NBUF + n_groups * _NBUF, _CHUNKS_PER_W):
        g = j + _LA
        emit(j, j % _NBUF, g if g < _CHUNKS_PER_W else None, True)

    for b in range(_NBUF):
        wait_stores(b)


@jax.jit
def _embed(x_lin, table_scaled):
    mesh = plsc.VectorSubcoreMesh(core_axis_name="c", subcore_axis_name="s")
    run = pl.kernel(
        _gather_body,
        out_type=jax.ShapeDtypeStruct((_SEQ, _SP, _DP), jnp.float32),
        mesh=mesh,
        scratch_types=[
            pltpu.VMEM((_IDX_PER_W,), jnp.int32),
            pltpu.VMEM((_NBUF, _CIDX, _D), jnp.float32),
            pltpu.SemaphoreType.DMA((_NBUF,)),
            pltpu.SemaphoreType.DMA((_NBUF,)),
        ],
        compiler_params=pltpu.CompilerParams(use_tc_tiling_on_sc=False),
    )
    return run(x_lin, table_scaled)


def kernel(x, embed_weight):
    x_lin = x.reshape(_SEQ * _SLEN).astype(jnp.int32)
    z = _embed(x_lin, embed_weight * _SCALE)
    return z[:, :_SLEN, :_D]


# speedup vs baseline: 1.3634x; 1.3406x over previous
import jax
import jax.numpy as jnp
from jax import lax
from jax.experimental import pallas as pl
from jax.experimental.pallas import tpu as pltpu
from jax.experimental.pallas import tpu_sc as plsc

_D = 64
_DP = 128
_SP = 56
_SCALE = float(_D) ** 0.5

_NW = 32
_SEQ = 16384
_SLEN = 50
_SEQ_PER_W = _SEQ // _NW
_CSEQ = 4
_CIDX = _CSEQ * _SLEN
_CHUNKS_PER_W = _SEQ_PER_W // _CSEQ
_IDX_PER_W = _SEQ_PER_W * _SLEN
_NBUF = 4
_LA = 2
_G_SPLITS = ((0, 96), (96, 104))


def _gather_body(x_hbm, table_hbm, out_hbm, idx_v, grow_v, gsems, ssems):
    wid = lax.axis_index("s") * 2 + lax.axis_index("c")

    pltpu.sync_copy(x_hbm.at[pl.ds(wid * _IDX_PER_W, _IDX_PER_W)], idx_v)

    seq_base = wid * _SEQ_PER_W

    def start_gather(g, b):
        for off, n in _G_SPLITS:
            pltpu.make_async_copy(
                table_hbm.at[idx_v.at[pl.ds(g * _CIDX + off, n)]],
                grow_v.at[b, pl.ds(off, n)],
                gsems.at[b],
            ).start()

    def wait_gather(b):
        for off, n in _G_SPLITS:
            pltpu.make_async_copy(
                table_hbm.at[idx_v.at[pl.ds(off, n)]],
                grow_v.at[b, pl.ds(off, n)],
                gsems.at[b],
            ).wait()

    def start_stores(g, b):
        for k in range(_CSEQ):
            pltpu.make_async_copy(
                grow_v.at[b, pl.ds(k * _SLEN, _SLEN)],
                out_hbm.at[seq_base + g * _CSEQ + k, pl.ds(0, _SLEN),
                           pl.ds(0, _D)],
                ssems.at[b],
            ).start()

    def wait_stores(b):
        for k in range(_CSEQ):
            pltpu.make_async_copy(
                grow_v.at[b, pl.ds(k * _SLEN, _SLEN)],
                out_hbm.at[seq_base, pl.ds(0, _SLEN), pl.ds(0, _D)],
                ssems.at[b],
            ).wait()


    def scale(b):
        def body(i, _):
            r = i * 2
            for k in range(2):
                for c in range(_D // 16):
                    sl = pl.ds(c * 16, 16)
                    grow_v[b, r + k, sl] = grow_v[b, r + k, sl] * _SCALE
            return 0

        lax.fori_loop(0, _CIDX // 2, body, 0)

    def emit(j, b, g, need_store_wait):
        wait_gather(b)
        scale(b)
        start_stores(j, b)
        if g is not None:
            b2 = (b + _LA) % _NBUF
            if need_store_wait:
                wait_stores(b2)
            start_gather(g, b2)

    for g in range(_LA):
        start_gather(g, g % _NBUF)

    for j in range(_NBUF):
        emit(j, j % _NBUF, j + _LA, j + _LA >= _NBUF)

    n_groups = (_CHUNKS_PER_W - _NBUF - _LA) // _NBUF

    def steady(t, _):
        j0 = _NBUF + t * _NBUF
        for i in range(_NBUF):
            emit(j0 + i, i, j0 + i + _LA, True)
        return 0

    lax.fori_loop(0, n_groups, steady, 0)

    for j in range(_NBUF + n_groups * _NBUF, _CHUNKS_PER_W):
        g = j + _LA
        emit(j, j % _NBUF, g if g < _CHUNKS_PER_W else None, True)

    for b in range(_NBUF):
        wait_stores(b)


@jax.jit
def _embed(x_lin, table_scaled):
    mesh = plsc.VectorSubcoreMesh(core_axis_name="c", subcore_axis_name="s")
    run = pl.kernel(
        _gather_body,
        out_type=jax.ShapeDtypeStruct((_SEQ, _SP, _DP), jnp.float32),
        mesh=mesh,
        scratch_types=[
            pltpu.VMEM((_IDX_PER_W,), jnp.int32),
            pltpu.VMEM((_NBUF, _CIDX, _D), jnp.float32),
            pltpu.SemaphoreType.DMA((_NBUF,)),
            pltpu.SemaphoreType.DMA((_NBUF,)),
        ],
        compiler_params=pltpu.CompilerParams(use_tc_tiling_on_sc=False),
    )
    return run(x_lin, table_scaled)


def kernel(x, embed_weight):
    x_lin = x.reshape(_SEQ * _SLEN).astype(jnp.int32)
    z = _embed(x_lin, embed_weight)
    return z[:, :_SLEN, :_D]


# speedup vs baseline: 1.3665x; 1.0023x over previous
import jax
import jax.numpy as jnp
from jax import lax
from jax.experimental import pallas as pl
from jax.experimental.pallas import tpu as pltpu
from jax.experimental.pallas import tpu_sc as plsc

_D = 64
_DP = 128
_SP = 56
_SCALE = float(_D) ** 0.5

_NW = 32
_SEQ = 16384
_SLEN = 50
_SEQ_PER_W = _SEQ // _NW
_CSEQ = 4
_CIDX = _CSEQ * _SLEN
_CHUNKS_PER_W = _SEQ_PER_W // _CSEQ
_IDX_PER_W = _SEQ_PER_W * _SLEN
_NBUF = 6
_LA = 3
_G_SPLITS = ((0, 96), (96, 104))


def _gather_body(x_hbm, table_hbm, out_hbm, idx_v, grow_v, gsems, ssems):
    wid = lax.axis_index("s") * 2 + lax.axis_index("c")

    pltpu.sync_copy(x_hbm.at[pl.ds(wid * _IDX_PER_W, _IDX_PER_W)], idx_v)

    seq_base = wid * _SEQ_PER_W

    def start_gather(g, b):
        for off, n in _G_SPLITS:
            pltpu.make_async_copy(
                table_hbm.at[idx_v.at[pl.ds(g * _CIDX + off, n)]],
                grow_v.at[b, pl.ds(off, n)],
                gsems.at[b],
            ).start()

    def wait_gather(b):
        for off, n in _G_SPLITS:
            pltpu.make_async_copy(
                table_hbm.at[idx_v.at[pl.ds(off, n)]],
                grow_v.at[b, pl.ds(off, n)],
                gsems.at[b],
            ).wait()

    def start_stores(g, b):
        for k in range(_CSEQ):
            pltpu.make_async_copy(
                grow_v.at[b, pl.ds(k * _SLEN, _SLEN)],
                out_hbm.at[seq_base + g * _CSEQ + k, pl.ds(0, _SLEN),
                           pl.ds(0, _D)],
                ssems.at[b],
            ).start()

    def wait_stores(b):
        for k in range(_CSEQ):
            pltpu.make_async_copy(
                grow_v.at[b, pl.ds(k * _SLEN, _SLEN)],
                out_hbm.at[seq_base, pl.ds(0, _SLEN), pl.ds(0, _D)],
                ssems.at[b],
            ).wait()


    def scale(b):
        def body(i, _):
            r = i * 4
            for k in range(4):
                for c in range(_D // 16):
                    sl = pl.ds(c * 16, 16)
                    grow_v[b, r + k, sl] = grow_v[b, r + k, sl] * _SCALE
            return 0

        lax.fori_loop(0, _CIDX // 4, body, 0)

    def emit(j, b, g, need_store_wait):
        wait_gather(b)
        scale(b)
        start_stores(j, b)
        if g is not None:
            b2 = (b + _LA) % _NBUF
            if need_store_wait:
                wait_stores(b2)
            start_gather(g, b2)

    for g in range(_LA):
        start_gather(g, g % _NBUF)

    for j in range(_NBUF):
        emit(j, j % _NBUF, j + _LA, j + _LA >= _NBUF)

    n_groups = (_CHUNKS_PER_W - _NBUF - _LA) // _NBUF

    def steady(t, _):
        j0 = _NBUF + t * _NBUF
        for i in range(_NBUF):
            emit(j0 + i, i, j0 + i + _LA, True)
        return 0

    lax.fori_loop(0, n_groups, steady, 0)

    for j in range(_NBUF + n_groups * _NBUF, _CHUNKS_PER_W):
        g = j + _LA
        emit(j, j % _NBUF, g if g < _CHUNKS_PER_W else None, True)

    for b in range(_NBUF):
        wait_stores(b)


@jax.jit
def _embed(x_lin, table_scaled):
    mesh = plsc.VectorSubcoreMesh(core_axis_name="c", subcore_axis_name="s")
    run = pl.kernel(
        _gather_body,
        out_type=jax.ShapeDtypeStruct((_SEQ, _SP, _DP), jnp.float32),
        mesh=mesh,
        scratch_types=[
            pltpu.VMEM((_IDX_PER_W,), jnp.int32),
            pltpu.VMEM((_NBUF, _CIDX, _D), jnp.float32),
            pltpu.SemaphoreType.DMA((_NBUF,)),
            pltpu.SemaphoreType.DMA((_NBUF,)),
        ],
        compiler_params=pltpu.CompilerParams(use_tc_tiling_on_sc=False),
    )
    return run(x_lin, table_scaled)


def kernel(x, embed_weight):
    x_lin = x.reshape(_SEQ * _SLEN).astype(jnp.int32)
    z = _embed(x_lin, embed_weight)
    return z[:, :_SLEN, :_D]
